# baseline (speedup 1.0000x reference)
import functools

import jax
import jax.numpy as jnp
from jax import lax
from jax.experimental import pallas as pl
from jax.experimental.pallas import tpu as pltpu
from jax.experimental.pallas import tpu_sc as plsc

_K = 1024
_D = 256
_BETA = 0.25
_NB = 16
_T = 1024
_NTOK = _NB * _T

_NW = 32
_RPW = _NTOK // _NW
_BUF = 32


def _vq_body(flat_ref, cb_ref, loss_ref, q_ref, perp_ref, idx_ref,
             cnt_ref, acc_ref):
    b = pl.program_id(0)

    @pl.when(b == 0)
    def _init():
        cnt_ref[...] = jnp.zeros_like(cnt_ref)
        acc_ref[0] = 0.0

    fb = flat_ref[...]
    cb = cb_ref[...]

    xsq = jnp.sum(fb * fb, axis=1, keepdims=True)
    cnorm = jnp.sum(cb * cb, axis=1)
    mm = jax.lax.dot_general(fb, cb, (((1,), (1,)), ((), ())),
                             preferred_element_type=jnp.float32)
    d = (xsq + cnorm[None, :]) - 2.0 * mm

    m = jnp.min(d, axis=1, keepdims=True)
    kio = jax.lax.broadcasted_iota(
        jnp.int32, (_T, _K), 1).astype(jnp.float32)
    idx = jnp.min(jnp.where(d == m, kio, float(_K)),
                  axis=1, keepdims=True)
    idx_ref[...] = idx.astype(jnp.int32)

    e = (kio == idx).astype(jnp.float32)
    cnt_ref[...] += jnp.sum(e, axis=0, keepdims=True)

    q = jax.lax.dot_general(e, cb, (((1,), (0,)), ((), ())),
                            preferred_element_type=jnp.float32)
    q_ref[...] = fb + (q - fb)

    acc_ref[0] += jnp.sum(m)

    @pl.when(b == _NB - 1)
    def _fin():
        mse = acc_ref[0] / (_NTOK * _D)
        loss_ref[...] = jnp.reshape((1.0 + _BETA) * mse, (1, 1))
        avg = cnt_ref[...] * (1.0 / _NTOK)
        ent = jnp.sum(avg * jnp.log(avg + 1e-10))
        perp_ref[...] = jnp.reshape(jnp.exp(-ent), (1, 1))


def _vq_call(flat, codebook):
    return pl.pallas_call(
        _vq_body,
        grid=(_NB,),
        in_specs=[
            pl.BlockSpec((_T, _D), lambda b: (b, 0)),
            pl.BlockSpec((_K, _D), lambda b: (0, 0)),
        ],
        out_specs=[
            pl.BlockSpec((1, 1), lambda b: (0, 0)),
            pl.BlockSpec((_T, _D), lambda b: (b, 0)),
            pl.BlockSpec((1, 1), lambda b: (0, 0)),
            pl.BlockSpec((_T, 1), lambda b: (b, 0)),
        ],
        out_shape=[
            jax.ShapeDtypeStruct((1, 1), jnp.float32),
            jax.ShapeDtypeStruct((_NTOK, _D), jnp.float32),
            jax.ShapeDtypeStruct((1, 1), jnp.float32),
            jax.ShapeDtypeStruct((_NTOK, 1), jnp.int32),
        ],
        scratch_shapes=[
            pltpu.VMEM((1, _K), jnp.float32),
            pltpu.SMEM((1,), jnp.float32),
        ],
        compiler_params=pltpu.CompilerParams(
            dimension_semantics=("arbitrary",),
        ),
    )(flat, codebook)


_sc_mesh = plsc.VectorSubcoreMesh(core_axis_name="c", subcore_axis_name="s")


@functools.partial(
    pl.kernel,
    mesh=_sc_mesh,
    out_type=jax.ShapeDtypeStruct((_NTOK * _K,), jnp.float32),
    scratch_types=[
        pltpu.VMEM((_RPW,), jnp.int32),
        pltpu.VMEM((_BUF * _K,), jnp.float32),
        pltpu.VMEM((128,), jnp.float32),
        pltpu.VMEM((4, 128), jnp.int32),
        pltpu.SemaphoreType.DMA,
    ],
)
def _sc_enc(idx_hbm, out_hbm, idxv, zbuf, ones_v, posbuf, sem):
    wid = lax.axis_index("s") * 2 + lax.axis_index("c")
    base = wid * _RPW
    pltpu.sync_copy(idx_hbm.at[pl.ds(base, _RPW)], idxv)

    zero16 = jnp.zeros((16,), jnp.float32)
    one16 = jnp.ones((16,), jnp.float32)
    lio = lax.iota(jnp.int32, 16)

    def _zf(i, carry):
        zbuf[pl.ds(i * 16, 16)] = zero16
        return carry
    lax.fori_loop(0, _BUF * _K // 16, _zf, 0)
    for sub in range(8):
        ones_v[pl.ds(sub * 16, 16)] = one16

    for bufi in range(_RPW // _BUF):
        pltpu.sync_copy(
            zbuf,
            out_hbm.at[pl.ds((base + bufi * _BUF) * _K, _BUF * _K)])

    for j in range(4):
        for sub in range(8):
            iv = idxv[pl.ds(j * 128 + sub * 16, 16)]
            rows = jnp.full((16,), base + j * 128 + sub * 16, jnp.int32) + lio
            posbuf[j, pl.ds(sub * 16, 16)] = rows * _K + iv
    for j in range(4):
        pltpu.async_copy(ones_v, out_hbm.at[posbuf.at[j]], sem).wait()


def kernel(encoder_output, codebook):
    b, c, h, w = encoder_output.shape
    flat = jnp.transpose(encoder_output, (0, 2, 3, 1)).reshape(-1, c)
    loss, q_st, perp, idxo = _vq_call(flat, codebook)
    enc = _sc_enc(idxo.reshape(_NTOK)).reshape(_NTOK, _K)
    q_bchw = jnp.transpose(q_st.reshape(b, h, w, c), (0, 3, 1, 2))
    return (loss[0, 0], q_bchw, perp[0, 0], enc)


# speedup vs baseline: 1.1039x; 1.1039x over previous
import jax
import jax.numpy as jnp
from jax.experimental import pallas as pl
from jax.experimental.pallas import tpu as pltpu

_K = 1024
_D = 256
_BETA = 0.25
_NB = 16
_T = 1024
_NTOK = _NB * _T


def _vq_body(flat_ref, cb_ref, loss_ref, q_ref, perp_ref, enc_ref,
             cnt_ref, acc_ref):
    b = pl.program_id(0)

    @pl.when(b == 0)
    def _init():
        cnt_ref[...] = jnp.zeros_like(cnt_ref)
        acc_ref[0] = 0.0

    fb = jnp.transpose(flat_ref[...])
    cb = cb_ref[...]

    xsq = jnp.sum(fb * fb, axis=1, keepdims=True)
    cnorm = jnp.sum(cb * cb, axis=1)
    mm = jax.lax.dot_general(fb, cb, (((1,), (1,)), ((), ())),
                             preferred_element_type=jnp.float32)
    d = (xsq + cnorm[None, :]) - 2.0 * mm

    m = jnp.min(d, axis=1, keepdims=True)
    kio = jax.lax.broadcasted_iota(
        jnp.int32, (_T, _K), 1).astype(jnp.float32)
    idx = jnp.min(jnp.where(d == m, kio, float(_K)),
                  axis=1, keepdims=True)

    e = (kio == idx).astype(jnp.float32)
    enc_ref[...] = e
    cnt_ref[...] += jnp.sum(e, axis=0, keepdims=True)

    q = jax.lax.dot_general(e, cb, (((1,), (0,)), ((), ())),
                            preferred_element_type=jnp.float32)
    q_ref[...] = jnp.transpose(fb + (q - fb))

    acc_ref[0] += jnp.sum(m)

    @pl.when(b == _NB - 1)
    def _fin():
        mse = acc_ref[0] / (_NTOK * _D)
        loss_ref[...] = jnp.reshape((1.0 + _BETA) * mse, (1, 1))
        avg = cnt_ref[...] * (1.0 / _NTOK)
        ent = jnp.sum(avg * jnp.log(avg + 1e-10))
        perp_ref[...] = jnp.reshape(jnp.exp(-ent), (1, 1))


def _vq_call(flat, codebook):
    return pl.pallas_call(
        _vq_body,
        grid=(_NB,),
        in_specs=[
            pl.BlockSpec((_D, _T), lambda b: (b, 0)),
            pl.BlockSpec((_K, _D), lambda b: (0, 0)),
        ],
        out_specs=[
            pl.BlockSpec((1, 1), lambda b: (0, 0)),
            pl.BlockSpec((_D, _T), lambda b: (b, 0)),
            pl.BlockSpec((1, 1), lambda b: (0, 0)),
            pl.BlockSpec((_T, _K), lambda b: (b, 0)),
        ],
        out_shape=[
            jax.ShapeDtypeStruct((1, 1), jnp.float32),
            jax.ShapeDtypeStruct((_NB * _D, _T), jnp.float32),
            jax.ShapeDtypeStruct((1, 1), jnp.float32),
            jax.ShapeDtypeStruct((_NTOK, _K), jnp.float32),
        ],
        scratch_shapes=[
            pltpu.VMEM((1, _K), jnp.float32),
            pltpu.SMEM((1,), jnp.float32),
        ],
        compiler_params=pltpu.CompilerParams(
            dimension_semantics=("arbitrary",),
        ),
    )(flat, codebook)


def kernel(encoder_output, codebook):
    b, c, h, w = encoder_output.shape
    x2d = encoder_output.reshape(b * c, h * w)
    loss, q_st, perp, enc = _vq_call(x2d, codebook)
    return (loss[0, 0], q_st.reshape(b, c, h, w), perp[0, 0], enc)


# speedup vs baseline: 3.4294x; 3.1066x over previous
import jax
import jax.numpy as jnp
from jax.experimental import pallas as pl
from jax.experimental.pallas import tpu as pltpu

_K = 1024
_D = 256
_BETA = 0.25
_NB = 16
_T = 1024
_NTOK = _NB * _T


def _vq_body(flat_ref, cb_ref, loss_ref, q_ref, perp_ref, enc_ref,
             cnt_ref, acc_ref):
    b = pl.program_id(0)

    @pl.when(b == 0)
    def _init():
        cnt_ref[...] = jnp.zeros_like(cnt_ref)
        acc_ref[0] = 0.0

    fb = flat_ref[...]
    cb = cb_ref[...]

    xsq = jnp.sum(fb * fb, axis=1, keepdims=True)
    cnorm = jnp.sum(cb * cb, axis=1)
    mm = jax.lax.dot_general(fb, cb, (((1,), (1,)), ((), ())),
                             preferred_element_type=jnp.float32)
    d = (xsq + cnorm[None, :]) - 2.0 * mm

    m = jnp.min(d, axis=1, keepdims=True)
    kio = jax.lax.broadcasted_iota(
        jnp.int32, (1, _K), 1).astype(jnp.float32)
    idx = jnp.min(jnp.where(d == m, kio, float(_K)),
                  axis=1, keepdims=True)

    e = (kio == idx).astype(jnp.float32)
    enc_ref[...] = e
    cnt_ref[...] += jnp.sum(e, axis=0, keepdims=True)

    q = jax.lax.dot_general(e, cb, (((1,), (0,)), ((), ())),
                            preferred_element_type=jnp.float32)
    q_ref[...] = fb + (q - fb)

    acc_ref[0] += jnp.sum(m)

    @pl.when(b == _NB - 1)
    def _fin():
        mse = acc_ref[0] / (_NTOK * _D)
        loss_ref[...] = jnp.reshape((1.0 + _BETA) * mse, (1, 1))
        avg = cnt_ref[...] * (1.0 / _NTOK)
        ent = jnp.sum(avg * jnp.log(avg + 1e-10))
        perp_ref[...] = jnp.reshape(jnp.exp(-ent), (1, 1))


def _vq_call(flat, codebook):
    return pl.pallas_call(
        _vq_body,
        grid=(_NB,),
        in_specs=[
            pl.BlockSpec((_T, _D), lambda b: (b, 0)),
            pl.BlockSpec((_K, _D), lambda b: (0, 0)),
        ],
        out_specs=[
            pl.BlockSpec((1, 1), lambda b: (0, 0)),
            pl.BlockSpec((_T, _D), lambda b: (b, 0)),
            pl.BlockSpec((1, 1), lambda b: (0, 0)),
            pl.BlockSpec((_T, _K), lambda b: (b, 0)),
        ],
        out_shape=[
            jax.ShapeDtypeStruct((1, 1), jnp.float32),
            jax.ShapeDtypeStruct((_NTOK, _D), jnp.float32),
            jax.ShapeDtypeStruct((1, 1), jnp.float32),
            jax.ShapeDtypeStruct((_NTOK, _K), jnp.float32),
        ],
        scratch_shapes=[
            pltpu.VMEM((1, _K), jnp.float32),
            pltpu.SMEM((1,), jnp.float32),
        ],
        compiler_params=pltpu.CompilerParams(
            dimension_semantics=("arbitrary",),
        ),
    )(flat, codebook)


def kernel(encoder_output, codebook):
    b, c, h, w = encoder_output.shape
    flat = jnp.transpose(encoder_output, (0, 2, 3, 1)).reshape(-1, c)
    loss, q_st, perp, enc = _vq_call(flat, codebook)
    q_bchw = jnp.transpose(q_st.reshape(b, h, w, c), (0, 3, 1, 2))
    return (loss[0, 0], q_bchw, perp[0, 0], enc)
